# baseline (speedup 1.0000x reference)
import functools
import math

import jax
import jax.numpy as jnp
from jax import lax
from jax.experimental import pallas as pl
from jax.experimental.pallas import tpu as pltpu
from jax.experimental.pallas import tpu_sc as plsc

S = 200
B = 1024
D = 64
SCALE = 8.0

NC = 2
NS = 16
NW = NC * NS
BGRP = 4
SGRP = NW // BGRP
S_PER = S // SGRP
B_PER = B // BGRP
K_PER = B_PER // 2
LANES = 16
NBLK = B_PER // LANES


def _make_pe(d_model, max_len):
    position = jnp.arange(0, max_len, dtype=jnp.float32)[:, None]
    div_term = jnp.exp(
        jnp.arange(0, d_model, 2, dtype=jnp.float32) * (-math.log(10000.0) / d_model)
    )
    pe = jnp.zeros((max_len, d_model), dtype=jnp.float32)
    pe = pe.at[:, 0::2].set(jnp.sin(position * div_term))
    pe = pe.at[:, 1::2].set(jnp.cos(position * div_term))
    return pe


@functools.partial(
    pl.kernel,
    mesh=plsc.VectorSubcoreMesh(core_axis_name="c", subcore_axis_name="s"),
    compiler_params=pltpu.CompilerParams(use_tc_tiling_on_sc=True),
    out_type=jax.ShapeDtypeStruct((S, B // 2, 2 * D), jnp.float32),
    scratch_types=[
        pltpu.VMEM((B_PER,), jnp.int32),
        pltpu.VMEM((2, 128), jnp.int32),
        pltpu.VMEM((B_PER, 2 * D), jnp.float32),
        pltpu.VMEM((K_PER, 2 * D), jnp.float32),
        pltpu.VMEM((2 * D,), jnp.float32),
        pltpu.SemaphoreType.DMA,
    ],
)
def _emb_kernel(xt_hbm, pe_hbm, tab_hbm, out_hbm, idx_v, j_v, g_v, o_v,
                pe_v, sem):
    wid = lax.axis_index("s") * NC + lax.axis_index("c")
    sgrp = wid // BGRP
    bq = wid % BGRP
    s_lo = sgrp * S_PER
    r0 = bq * 2
    k0 = bq * K_PER

    def body(i, carry):
        s = s_lo + i
        pltpu.sync_copy(xt_hbm.at[s, r0], idx_v.at[pl.ds(0, 128)])
        pltpu.sync_copy(xt_hbm.at[s, r0 + 1], idx_v.at[pl.ds(128, 128)])
        pltpu.sync_copy(pe_hbm.at[s], pe_v)
        for a in range(2):
            for t in range(128 // LANES):
                j_v[a, pl.ds(LANES * t, LANES)] = lax.shift_right_logical(
                    idx_v[pl.ds(128 * a + LANES * t, LANES)], 1
                )
        cp0 = pltpu.async_copy(tab_hbm.at[j_v.at[0]], g_v.at[pl.ds(0, 128)], sem)
        cp1 = pltpu.async_copy(tab_hbm.at[j_v.at[1]], g_v.at[pl.ds(128, 128)], sem)
        cp0.wait()
        cp1.wait()
        pe_regs = [pe_v[pl.ds(LANES * c, LANES)] for c in range(2 * D // LANES)]

        def block(blk, rcarry):
            offv = (idx_v[pl.ds(LANES * blk, LANES)] & 1) * D
            p_base = LANES * blk
            q_base = 8 * blk
            for u in range(8):
                q = q_base + u
                p0 = p_base + 2 * u
                off0 = offv[2 * u]
                off1 = offv[2 * u + 1]
                for c in range(D // LANES):
                    o_v[q, pl.ds(LANES * c, LANES)] = (
                        g_v[p0, pl.ds(off0 + LANES * c, LANES)] * SCALE
                        + pe_regs[c]
                    )
                    o_v[q, pl.ds(D + LANES * c, LANES)] = (
                        g_v[p0 + 1, pl.ds(off1 + LANES * c, LANES)] * SCALE
                        + pe_regs[4 + c]
                    )
            return rcarry

        lax.fori_loop(0, NBLK, block, 0)
        pltpu.sync_copy(o_v, out_hbm.at[s, pl.ds(k0, K_PER)])
        return carry

    lax.fori_loop(0, S_PER, body, 0)


def kernel(x, emb_table):
    xt = jnp.reshape(jnp.transpose(x.astype(jnp.int32), (1, 0)), (S, 8, 128))
    tab_wide = jnp.reshape(emb_table, (emb_table.shape[0] // 2, 2 * D))
    pe = _make_pe(D, S)
    pe_wide = jnp.concatenate([pe, pe], axis=1)
    out = _emb_kernel(xt, pe_wide, tab_wide)
    return jnp.reshape(out, (S, B, D))
